# baseline (speedup 1.0000x reference)
import functools

import jax
import jax.numpy as jnp
from jax import lax
from jax.experimental import pallas as pl
from jax.experimental.pallas import tpu as pltpu
from jax.experimental.pallas import tpu_sc as plsc

_L = 16


def _gather_call(B, V, D, NC, NS):
    NW = NC * NS
    n = B // NW
    G = n // _L
    mesh = plsc.VectorSubcoreMesh(core_axis_name="c", subcore_axis_name="s")

    @functools.partial(
        pl.kernel,
        mesh=mesh,
        out_type=jax.ShapeDtypeStruct((B // _L, _L, D), jnp.float32),
        scratch_types=[
            pltpu.VMEM((G, _L), jnp.int32),
            pltpu.VMEM((G, _L, D), jnp.float32),
            [pltpu.SemaphoreType.DMA] * 4,
        ],
        compiler_params=pltpu.CompilerParams(needs_layout_passes=False),
    )
    def body(idx_hbm, table_hbm, out_hbm, idx_v, rows_v, sems):
        wid = lax.axis_index("s") * NC + lax.axis_index("c")
        base = wid * G
        pltpu.sync_copy(idx_hbm.at[wid], idx_v)
        iota = lax.iota(jnp.int32, _L)
        NCHUNK = 4
        H = G // NCHUNK

        def group(g, _, sem=None):
            vec = idx_v[g]
            for j in range(_L):
                i = jnp.sum(jnp.where(iota == j, vec, 0))
                pltpu.make_async_copy(
                    table_hbm.at[i >> 4, i & 15],
                    rows_v.at[g, j],
                    sem,
                ).start()
            return 0

        for c in range(NCHUNK):
            lax.fori_loop(
                c * H, (c + 1) * H, functools.partial(group, sem=sems[c]), 0
            )
        for c in range(NCHUNK):
            pltpu.make_async_copy(
                table_hbm.at[pl.ds(0, H)], rows_v.at[pl.ds(c * H, H)], sems[c]
            ).wait()
            pltpu.sync_copy(
                rows_v.at[pl.ds(c * H, H)], out_hbm.at[pl.ds(base + c * H, H)]
            )

    return body


def kernel(e, params):
    B = e.shape[0]
    V, D = params.shape
    info = plsc.get_sparse_core_info()
    NC, NS = info.num_cores, info.num_subcores
    NW = NC * NS
    idx = e.astype(jnp.int32).reshape(NW, (B // NW) // _L, _L)
    table3 = params.reshape(V // _L, _L, D)
    out = _gather_call(B, V, D, NC, NS)(idx, table3)
    return out.reshape(B, D)
